# Initial kernel scaffold; baseline (speedup 1.0000x reference)
import jax
import jax.numpy as jnp
from jax.experimental import pallas as pl


def kernel(x, edge_index, edge_attr, fc11_w, fc11_b, fc21_w, fc21_b, c21_rel_w, c21_rel_b, c21_root_w, c22_rel_w, c22_rel_b, c22_root_w):
    raise NotImplementedError("write your pallas kernel here")


# speedup vs baseline: 4.1005x; 4.1005x over previous
import functools
import math

import jax
import jax.numpy as jnp
from jax import lax
from jax.experimental import pallas as pl
from jax.experimental.pallas import tpu as pltpu
from jax.experimental.pallas import tpu_sc as plsc

N = 4096
E = 131072
F2 = 8
F3 = 16
NC = 2
NS = 16
NW = NC * NS
EPW = E // NW
CHUNK = 128
NCHUNK = EPW // CHUNK
LOG_EPS = math.log(1e-10)
BLK = 256

_f32 = jnp.float32
_i32 = jnp.int32


def _wid():
    return lax.axis_index("s") * NC + lax.axis_index("c")


def _zero_1d(ref, n):
    z = jnp.zeros((16,), ref.dtype)

    def body(i, _):
        ref[pl.ds(i * 16, 16)] = z
        return 0

    lax.fori_loop(0, n // 16, body, 0)


def _keys_into(key_v, s_v, d_v):
    for j in range(NCHUNK):
        for c in range(CHUNK // 16):
            sl = pl.ds(j * CHUNK + c * 16, 16)
            key_v[j, pl.ds(c * 16, 16)] = s_v[sl] * N + d_v[sl]


def _pre_body(x_ref, f11w_ref, f11b_ref, f21w_ref, f21b_ref, param_ref, h1_ref):
    xv = x_ref[...]
    p = xv @ f11w_ref[...] + f11b_ref[...][None, :]
    param_ref[...] = jnp.maximum(p, 0.0)
    z = xv @ f21w_ref[...] + f21b_ref[...][None, :]
    h1_ref[...] = jnp.where(z >= 0.0, z, 0.01 * z)


def _pre(x, f11w, f11b, f21w, f21b):
    return pl.pallas_call(
        _pre_body,
        out_shape=(jax.ShapeDtypeStruct((1, 1), _f32),
                   jax.ShapeDtypeStruct((1, N), _f32)),
    )(x, f11w, f11b, f21w, f21b)


def _own_body(src_hbm, dst_hbm, tmp_out, s_v, d_v, key_v, e_v, sem):
    base = _wid() * EPW
    pltpu.sync_copy(src_hbm.at[pl.ds(base, EPW)], s_v)
    pltpu.sync_copy(dst_hbm.at[pl.ds(base, EPW)], d_v)
    _keys_into(key_v, s_v, d_v)
    iota = lax.iota(_i32, 16)

    def ebody(i, _):
        e_v[pl.ds(i * 16, 16)] = base + i * 16 + iota
        return 0

    lax.fori_loop(0, EPW // 16, ebody, 0)
    cps = [
        pltpu.async_copy(e_v.at[pl.ds(j * CHUNK, CHUNK)],
                         tmp_out.at[key_v.at[j]], sem)
        for j in range(NCHUNK)
    ]
    for cp in cps:
        cp.wait()


def _conv1_body(h1_hbm, src_hbm, dst_hbm, w_hbm, agg_out, deg_out,
                h1_v, agg_v, deg_v, s_v, d_v, w_v):
    wid = _wid()
    base = wid * EPW
    pltpu.sync_copy(h1_hbm, h1_v)
    pltpu.sync_copy(src_hbm.at[pl.ds(base, EPW)], s_v)
    pltpu.sync_copy(dst_hbm.at[pl.ds(base, EPW)], d_v)
    pltpu.sync_copy(w_hbm.at[pl.ds(base, EPW)], w_v)
    _zero_1d(agg_v, N)
    _zero_1d(deg_v, N)
    ones = jnp.ones((16,), _f32)

    def ebody(i, _):
        sl = pl.ds(i * 16, 16)
        s = s_v[sl]
        d = d_v[sl]
        wv = w_v[sl]
        hv = plsc.load_gather(h1_v, [s])
        plsc.addupdate_scatter(agg_v, [d], wv * hv)
        plsc.addupdate_scatter(deg_v, [d], ones)
        return 0

    lax.fori_loop(0, EPW // 16, ebody, 0)
    pltpu.sync_copy(agg_v, agg_out.at[wid])
    pltpu.sync_copy(deg_v, deg_out.at[wid])


def _mid_body(aggp_ref, degp_ref, h1_ref, rw_ref, rb_ref, tw_ref,
              h2T_ref, inv_ref):
    deg = jnp.sum(degp_ref[...], axis=0, keepdims=True)
    inv = 1.0 / jnp.maximum(deg, 1.0)
    inv_ref[...] = inv
    aggm = jnp.sum(aggp_ref[...], axis=0, keepdims=True) * inv
    h1 = h1_ref[...]
    zT = (lax.dot_general(rw_ref[...], aggm, (((0,), (0,)), ((), ())))
          + lax.dot_general(tw_ref[...], h1, (((0,), (0,)), ((), ())))
          + rb_ref[...][:, None])
    h2T_ref[...] = jnp.where(zT >= 0.0, zT, 0.01 * zT)


def _mid(agg1p, degp, h1row, rw, rb, tw):
    return pl.pallas_call(
        _mid_body,
        out_shape=(jax.ShapeDtypeStruct((F2, N), _f32),
                   jax.ShapeDtypeStruct((1, N), _f32)),
    )(agg1p, degp, h1row, rw, rb, tw)


def _conv2_body(h2T_hbm, src_hbm, dst_hbm, w_hbm, agg_out, *refs):
    h2T_f = refs[:F2]
    agg_f = refs[F2:2 * F2]
    s_v, d_v, w_v = refs[2 * F2:2 * F2 + 3]
    wid = _wid()
    base = wid * EPW
    for f in range(F2):
        pltpu.sync_copy(h2T_hbm.at[f], h2T_f[f])
        _zero_1d(agg_f[f], N)
    pltpu.sync_copy(src_hbm.at[pl.ds(base, EPW)], s_v)
    pltpu.sync_copy(dst_hbm.at[pl.ds(base, EPW)], d_v)
    pltpu.sync_copy(w_hbm.at[pl.ds(base, EPW)], w_v)

    def ebody(i, _):
        sl = pl.ds(i * 16, 16)
        s = s_v[sl]
        d = d_v[sl]
        wv = w_v[sl]
        for f in range(F2):
            hv = plsc.load_gather(h2T_f[f], [s])
            plsc.addupdate_scatter(agg_f[f], [d], wv * hv)
        return 0

    lax.fori_loop(0, EPW // 16, ebody, 0)
    for f in range(F2):
        pltpu.sync_copy(agg_f[f], agg_out.at[wid, f])


def _acc_body(tmp_hbm, src_hbm, dst_hbm, w_hbm, acc_out,
              acc_sh, zb_v, s_v, d_v, w_v, key_v, own_v, sem):
    cid = lax.axis_index("c")
    sid = lax.axis_index("s")
    base = (sid * NC + cid) * EPW
    seg = E // NS
    _zero_1d(zb_v, seg)
    pltpu.sync_copy(zb_v, acc_sh.at[pl.ds(sid * seg, seg)])
    plsc.subcore_barrier()
    pltpu.sync_copy(src_hbm.at[pl.ds(base, EPW)], s_v)
    pltpu.sync_copy(dst_hbm.at[pl.ds(base, EPW)], d_v)
    pltpu.sync_copy(w_hbm.at[pl.ds(base, EPW)], w_v)
    _keys_into(key_v, s_v, d_v)
    cps = [
        pltpu.async_copy(tmp_hbm.at[key_v.at[j]], own_v.at[j], sem)
        for j in range(NCHUNK)
    ]
    for cp in cps:
        cp.wait()
    for j in range(NCHUNK):
        pltpu.sync_copy(w_v.at[pl.ds(j * CHUNK, CHUNK)],
                        acc_sh.at[own_v.at[j]], add=True)
    plsc.subcore_barrier()
    pltpu.sync_copy(acc_sh.at[pl.ds(sid * seg, seg)],
                    acc_out.at[cid, pl.ds(sid * seg, seg)])


def _out_body(agg2p_ref, inv_ref, h2T_ref, rw_ref, rb_ref, tw_ref,
              accp_ref, p_ref, out_ref, outT_ref, lwp_ref):
    agg2 = jnp.sum(agg2p_ref[...], axis=0)
    aggm = agg2 * inv_ref[...]
    h2T = h2T_ref[...]
    rw = rw_ref[...]
    tw = tw_ref[...]
    rb = rb_ref[...]
    zT = (lax.dot_general(rw, aggm, (((0,), (0,)), ((), ())))
          + lax.dot_general(tw, h2T, (((0,), (0,)), ((), ())))
          + rb[:, None])
    outT_ref[...] = jnp.where(zT >= 0.0, zT, 0.01 * zT)
    z = (lax.dot_general(aggm, rw, (((0,), (0,)), ((), ())))
         + lax.dot_general(h2T, tw, (((0,), (0,)), ((), ())))
         + rb[None, :])
    out_ref[...] = jnp.where(z >= 0.0, z, 0.01 * z)
    wsum = accp_ref[0, :] + accp_ref[1, :]
    lwp_ref[...] = p_ref[0] * jnp.log(wsum + 1e-10)


def _outk(agg2p, invdeg, h2T, rw, rb, tw, accp, param):
    return pl.pallas_call(
        _out_body,
        out_shape=(jax.ShapeDtypeStruct((N, F3), _f32),
                   jax.ShapeDtypeStruct((F3, N), _f32),
                   jax.ShapeDtypeStruct((E,), _f32)),
        in_specs=[
            pl.BlockSpec(memory_space=pltpu.VMEM),
            pl.BlockSpec(memory_space=pltpu.VMEM),
            pl.BlockSpec(memory_space=pltpu.VMEM),
            pl.BlockSpec(memory_space=pltpu.VMEM),
            pl.BlockSpec(memory_space=pltpu.VMEM),
            pl.BlockSpec(memory_space=pltpu.VMEM),
            pl.BlockSpec(memory_space=pltpu.VMEM),
            pl.BlockSpec(memory_space=pltpu.SMEM),
        ],
    )(agg2p, invdeg, h2T, rw, rb, tw, accp, param.reshape(1))


def _dense_body(p_ref, rows_ref, colsT_ref, o_ref):
    c0 = p_ref[0] * LOG_EPS
    z = lax.dot_general(rows_ref[...], colsT_ref[...],
                        (((1,), (0,)), ((), ())),
                        preferred_element_type=_f32)
    o_ref[...] = 1.0 / (1.0 + jnp.exp(-(z + c0)))


def _dense(param, out, outT):
    return pl.pallas_call(
        _dense_body,
        grid=(N // BLK,),
        in_specs=[
            pl.BlockSpec(memory_space=pltpu.SMEM),
            pl.BlockSpec((BLK, F3), lambda i: (i, 0)),
            pl.BlockSpec((F3, N), lambda i: (0, 0)),
        ],
        out_specs=pl.BlockSpec((BLK, N), lambda i: (i, 0)),
        out_shape=jax.ShapeDtypeStruct((N, N), _f32),
    )(param.reshape(1), out, outT)


def _patch_body(conn_ref, outT_hbm, src_hbm, dst_hbm, tmp_hbm, lwp_hbm,
                *refs):
    outT_f = refs[:F3]
    s_v, d_v, key_v, own_v, lw_v, val_v, sem = refs[F3:F3 + 7]
    base = _wid() * EPW
    for f in range(F3):
        pltpu.sync_copy(outT_hbm.at[f], outT_f[f])
    pltpu.sync_copy(src_hbm.at[pl.ds(base, EPW)], s_v)
    pltpu.sync_copy(dst_hbm.at[pl.ds(base, EPW)], d_v)
    _keys_into(key_v, s_v, d_v)
    cps = [
        pltpu.async_copy(tmp_hbm.at[key_v.at[j]], own_v.at[j], sem)
        for j in range(NCHUNK)
    ]
    for cp in cps:
        cp.wait()
    cps = [
        pltpu.async_copy(lwp_hbm.at[own_v.at[j]],
                         lw_v.at[pl.ds(j * CHUNK, CHUNK)], sem)
        for j in range(NCHUNK)
    ]
    for cp in cps:
        cp.wait()

    def cbody(i, _):
        sl = pl.ds(i * 16, 16)
        s = s_v[sl]
        d = d_v[sl]
        acc = jnp.zeros((16,), _f32)
        for f in range(F3):
            acc = acc + (plsc.load_gather(outT_f[f], [s])
                         * plsc.load_gather(outT_f[f], [d]))
        z = acc + lw_v[sl]
        val_v[sl] = 1.0 / (1.0 + jnp.exp(-z))
        return 0

    lax.fori_loop(0, EPW // 16, cbody, 0)
    cps = [
        pltpu.async_copy(val_v.at[pl.ds(j * CHUNK, CHUNK)],
                         conn_ref.at[key_v.at[j]], sem)
        for j in range(NCHUNK)
    ]
    for cp in cps:
        cp.wait()


@functools.cache
def _sc():
    mesh = plsc.VectorSubcoreMesh(core_axis_name="c", subcore_axis_name="s",
                                  num_cores=NC, num_subcores=NS)
    cp = pltpu.CompilerParams(needs_layout_passes=False)
    own = pl.kernel(
        _own_body,
        out_type=jax.ShapeDtypeStruct((N * N,), _i32),
        mesh=mesh,
        compiler_params=cp,
        scratch_types=[
            pltpu.VMEM((EPW,), _i32), pltpu.VMEM((EPW,), _i32),
            pltpu.VMEM((NCHUNK, CHUNK), _i32), pltpu.VMEM((EPW,), _i32),
            pltpu.SemaphoreType.DMA,
        ],
    )
    conv1 = pl.kernel(
        _conv1_body,
        out_type=(jax.ShapeDtypeStruct((NW, N), _f32),
                  jax.ShapeDtypeStruct((NW, N), _f32)),
        mesh=mesh,
        compiler_params=cp,
        scratch_types=[
            pltpu.VMEM((N,), _f32), pltpu.VMEM((N,), _f32),
            pltpu.VMEM((N,), _f32),
            pltpu.VMEM((EPW,), _i32), pltpu.VMEM((EPW,), _i32),
            pltpu.VMEM((EPW,), _f32),
        ],
    )
    conv2 = pl.kernel(
        _conv2_body,
        out_type=jax.ShapeDtypeStruct((NW, F2, N), _f32),
        mesh=mesh,
        compiler_params=cp,
        scratch_types=[pltpu.VMEM((N,), _f32)] * (2 * F2) + [
            pltpu.VMEM((EPW,), _i32), pltpu.VMEM((EPW,), _i32),
            pltpu.VMEM((EPW,), _f32),
        ],
    )
    acc = pl.kernel(
        _acc_body,
        out_type=jax.ShapeDtypeStruct((NC, E), _f32),
        mesh=mesh,
        compiler_params=cp,
        scratch_types=[
            pltpu.VMEM_SHARED((E,), _f32),
            pltpu.VMEM((E // NS,), _f32),
            pltpu.VMEM((EPW,), _i32), pltpu.VMEM((EPW,), _i32),
            pltpu.VMEM((EPW,), _f32),
            pltpu.VMEM((NCHUNK, CHUNK), _i32),
            pltpu.VMEM((NCHUNK, CHUNK), _i32),
            pltpu.SemaphoreType.DMA,
        ],
    )
    patch = pl.kernel(
        _patch_body,
        out_type=(),
        mesh=mesh,
        compiler_params=cp,
        scratch_types=[pltpu.VMEM((N,), _f32)] * F3 + [
            pltpu.VMEM((EPW,), _i32), pltpu.VMEM((EPW,), _i32),
            pltpu.VMEM((NCHUNK, CHUNK), _i32),
            pltpu.VMEM((NCHUNK, CHUNK), _i32),
            pltpu.VMEM((EPW,), _f32), pltpu.VMEM((EPW,), _f32),
            pltpu.SemaphoreType.DMA,
        ],
    )
    return own, conv1, conv2, acc, patch


def kernel(x, edge_index, edge_attr, fc11_w, fc11_b, fc21_w, fc21_b,
           c21_rel_w, c21_rel_b, c21_root_w, c22_rel_w, c22_rel_b, c22_root_w):
    own, conv1, conv2, acc, patch = _sc()
    src = edge_index[0]
    dst = edge_index[1]
    param, h1row = _pre(x, fc11_w, fc11_b, fc21_w, fc21_b)
    h1 = h1row.reshape(N)
    tmp = own(src, dst)
    agg1p, degp = conv1(h1, src, dst, edge_attr)
    h2T, invdeg = _mid(agg1p, degp, h1row, c21_rel_w, c21_rel_b, c21_root_w)
    agg2p = conv2(h2T, src, dst, edge_attr)
    accp = acc(tmp, src, dst, edge_attr)
    out, outT, lwp = _outk(agg2p, invdeg, h2T, c22_rel_w, c22_rel_b,
                           c22_root_w, accp, param)
    dense = _dense(param, out, outT)
    flat_ref = jax.new_ref(dense.reshape(N * N))
    patch(flat_ref, outT, src, dst, tmp, lwp)
    conn = flat_ref[...].reshape(1, N, N)
    return conn, param


# speedup vs baseline: 5.3108x; 1.2951x over previous
import functools
import math

import jax
import jax.numpy as jnp
from jax import lax
from jax.experimental import pallas as pl
from jax.experimental.pallas import tpu as pltpu
from jax.experimental.pallas import tpu_sc as plsc

N = 4096
E = 131072
F2 = 8
F3 = 16
NC = 2
NS = 16
NW = NC * NS
EPW = E // NW
CHUNK = 128
NCHUNK = EPW // CHUNK
LOG_EPS = math.log(1e-10)
BLK = 256

_f32 = jnp.float32
_i32 = jnp.int32


def _wid():
    return lax.axis_index("s") * NC + lax.axis_index("c")


def _zero_1d(ref, n):
    z = jnp.zeros((16,), ref.dtype)

    def body(i, _):
        ref[pl.ds(i * 16, 16)] = z
        return 0

    lax.fori_loop(0, n // 16, body, 0)


def _keys_into(key_v, s_v, d_v):
    for j in range(NCHUNK):
        for c in range(CHUNK // 16):
            sl = pl.ds(j * CHUNK + c * 16, 16)
            key_v[j, pl.ds(c * 16, 16)] = s_v[sl] * N + d_v[sl]


def _pre_body(x_ref, f11w_ref, f11b_ref, f21w_ref, f21b_ref, param_ref, h1_ref):
    xv = x_ref[...]
    p = xv @ f11w_ref[...] + f11b_ref[...][None, :]
    param_ref[...] = jnp.maximum(p, 0.0)
    z = xv @ f21w_ref[...] + f21b_ref[...][None, :]
    h1_ref[...] = jnp.where(z >= 0.0, z, 0.01 * z)


def _pre(x, f11w, f11b, f21w, f21b):
    return pl.pallas_call(
        _pre_body,
        out_shape=(jax.ShapeDtypeStruct((1, 1), _f32),
                   jax.ShapeDtypeStruct((1, N), _f32)),
    )(x, f11w, f11b, f21w, f21b)


def _own_body(src_hbm, dst_hbm, tmp_out, s_v, d_v, key_v, e_v, sem):
    base = _wid() * EPW
    pltpu.sync_copy(src_hbm.at[pl.ds(base, EPW)], s_v)
    pltpu.sync_copy(dst_hbm.at[pl.ds(base, EPW)], d_v)
    _keys_into(key_v, s_v, d_v)
    iota = lax.iota(_i32, 16)

    def ebody(i, _):
        e_v[pl.ds(i * 16, 16)] = base + i * 16 + iota
        return 0

    lax.fori_loop(0, EPW // 16, ebody, 0)
    cps = [
        pltpu.async_copy(e_v.at[pl.ds(j * CHUNK, CHUNK)],
                         tmp_out.at[key_v.at[j]], sem)
        for j in range(NCHUNK)
    ]
    for cp in cps:
        cp.wait()


def _conv1_body(h1_hbm, src_hbm, dst_hbm, w_hbm, agg_out, deg_out,
                h1_v, agg_v, deg_v, s_v, d_v, w_v):
    wid = _wid()
    base = wid * EPW
    pltpu.sync_copy(h1_hbm, h1_v)
    pltpu.sync_copy(src_hbm.at[pl.ds(base, EPW)], s_v)
    pltpu.sync_copy(dst_hbm.at[pl.ds(base, EPW)], d_v)
    pltpu.sync_copy(w_hbm.at[pl.ds(base, EPW)], w_v)
    _zero_1d(agg_v, N)
    _zero_1d(deg_v, N)
    ones = jnp.ones((16,), _f32)

    def ebody(i, _):
        sl = pl.ds(i * 16, 16)
        s = s_v[sl]
        d = d_v[sl]
        wv = w_v[sl]
        hv = plsc.load_gather(h1_v, [s])
        plsc.addupdate_scatter(agg_v, [d], wv * hv)
        plsc.addupdate_scatter(deg_v, [d], ones)
        return 0

    lax.fori_loop(0, EPW // 16, ebody, 0)
    pltpu.sync_copy(agg_v, agg_out.at[wid])
    pltpu.sync_copy(deg_v, deg_out.at[wid])


def _mid_body(aggp_ref, degp_ref, h1_ref, rw_ref, rb_ref, tw_ref,
              h2T_ref, inv_ref):
    deg = jnp.sum(degp_ref[...], axis=0, keepdims=True)
    inv = 1.0 / jnp.maximum(deg, 1.0)
    inv_ref[...] = inv
    aggm = jnp.sum(aggp_ref[...], axis=0, keepdims=True) * inv
    h1 = h1_ref[...]
    zT = (lax.dot_general(rw_ref[...], aggm, (((0,), (0,)), ((), ())))
          + lax.dot_general(tw_ref[...], h1, (((0,), (0,)), ((), ())))
          + rb_ref[...][:, None])
    h2T_ref[...] = jnp.where(zT >= 0.0, zT, 0.01 * zT)


def _mid(agg1p, degp, h1row, rw, rb, tw):
    return pl.pallas_call(
        _mid_body,
        out_shape=(jax.ShapeDtypeStruct((F2, N), _f32),
                   jax.ShapeDtypeStruct((1, N), _f32)),
    )(agg1p, degp, h1row, rw, rb, tw)


def _conv2_body(h2T_hbm, src_hbm, dst_hbm, w_hbm, agg_out, *refs):
    h2T_f = refs[:F2]
    agg_f = refs[F2:2 * F2]
    s_v, d_v, w_v = refs[2 * F2:2 * F2 + 3]
    wid = _wid()
    base = wid * EPW
    for f in range(F2):
        pltpu.sync_copy(h2T_hbm.at[f], h2T_f[f])
        _zero_1d(agg_f[f], N)
    pltpu.sync_copy(src_hbm.at[pl.ds(base, EPW)], s_v)
    pltpu.sync_copy(dst_hbm.at[pl.ds(base, EPW)], d_v)
    pltpu.sync_copy(w_hbm.at[pl.ds(base, EPW)], w_v)

    def ebody(i, _):
        sl = pl.ds(i * 16, 16)
        s = s_v[sl]
        d = d_v[sl]
        wv = w_v[sl]
        for f in range(F2):
            hv = plsc.load_gather(h2T_f[f], [s])
            plsc.addupdate_scatter(agg_f[f], [d], wv * hv)
        return 0

    lax.fori_loop(0, EPW // 16, ebody, 0)
    for f in range(F2):
        pltpu.sync_copy(agg_f[f], agg_out.at[wid, f])


def _acc_body(tmp_hbm, src_hbm, dst_hbm, w_hbm, acc_out, own_out,
              acc_sh, zb_v, s_v, d_v, w_v, key_v, own_v, sem):
    cid = lax.axis_index("c")
    sid = lax.axis_index("s")
    base = (sid * NC + cid) * EPW
    seg = E // NS
    _zero_1d(zb_v, seg)
    pltpu.sync_copy(zb_v, acc_sh.at[pl.ds(sid * seg, seg)])
    plsc.subcore_barrier()
    pltpu.sync_copy(src_hbm.at[pl.ds(base, EPW)], s_v)
    pltpu.sync_copy(dst_hbm.at[pl.ds(base, EPW)], d_v)
    pltpu.sync_copy(w_hbm.at[pl.ds(base, EPW)], w_v)
    _keys_into(key_v, s_v, d_v)
    cps = [
        pltpu.async_copy(tmp_hbm.at[key_v.at[j]], own_v.at[j], sem)
        for j in range(NCHUNK)
    ]
    for cp in cps:
        cp.wait()
    for j in range(NCHUNK):
        pltpu.sync_copy(w_v.at[pl.ds(j * CHUNK, CHUNK)],
                        acc_sh.at[own_v.at[j]], add=True)
    pltpu.sync_copy(own_v, own_out.at[sid * NC + cid])
    plsc.subcore_barrier()
    pltpu.sync_copy(acc_sh.at[pl.ds(sid * seg, seg)],
                    acc_out.at[cid, pl.ds(sid * seg, seg)])


def _out_body(agg2p_ref, inv_ref, h2T_ref, rw_ref, rb_ref, tw_ref,
              accp_ref, p_ref, out_ref, outT_ref, lwp_ref):
    agg2 = jnp.sum(agg2p_ref[...], axis=0)
    aggm = agg2 * inv_ref[...]
    h2T = h2T_ref[...]
    rw = rw_ref[...]
    tw = tw_ref[...]
    rb = rb_ref[...]
    zT = (lax.dot_general(rw, aggm, (((0,), (0,)), ((), ())))
          + lax.dot_general(tw, h2T, (((0,), (0,)), ((), ())))
          + rb[:, None])
    outT_ref[...] = jnp.where(zT >= 0.0, zT, 0.01 * zT)
    z = (lax.dot_general(aggm, rw, (((0,), (0,)), ((), ())))
         + lax.dot_general(h2T, tw, (((0,), (0,)), ((), ())))
         + rb[None, :])
    out_ref[...] = jnp.where(z >= 0.0, z, 0.01 * z)
    wsum = accp_ref[0, :] + accp_ref[1, :]
    lwp_ref[...] = p_ref[0] * jnp.log(wsum + 1e-10)


def _outk(agg2p, invdeg, h2T, rw, rb, tw, accp, param):
    return pl.pallas_call(
        _out_body,
        out_shape=(jax.ShapeDtypeStruct((N, F3), _f32),
                   jax.ShapeDtypeStruct((F3, N), _f32),
                   jax.ShapeDtypeStruct((E,), _f32)),
        in_specs=[
            pl.BlockSpec(memory_space=pltpu.VMEM),
            pl.BlockSpec(memory_space=pltpu.VMEM),
            pl.BlockSpec(memory_space=pltpu.VMEM),
            pl.BlockSpec(memory_space=pltpu.VMEM),
            pl.BlockSpec(memory_space=pltpu.VMEM),
            pl.BlockSpec(memory_space=pltpu.VMEM),
            pl.BlockSpec(memory_space=pltpu.VMEM),
            pl.BlockSpec(memory_space=pltpu.SMEM),
        ],
    )(agg2p, invdeg, h2T, rw, rb, tw, accp, param.reshape(1))


def _dense_body(p_ref, rows_ref, colsT_ref, o_ref):
    c0 = p_ref[0] * LOG_EPS
    z = lax.dot_general(rows_ref[...], colsT_ref[...],
                        (((1,), (0,)), ((), ())),
                        preferred_element_type=_f32)
    v = 0.5 * jnp.tanh(0.5 * (z + c0)) + 0.5
    for t in range(BLK // 8):
        vt = v[t * 8:(t + 1) * 8, :]
        for c in range(N // 128):
            o_ref[t, c] = vt[:, c * 128:(c + 1) * 128]


def _dense(param, out, outT):
    return pl.pallas_call(
        _dense_body,
        grid=(N // BLK,),
        in_specs=[
            pl.BlockSpec(memory_space=pltpu.SMEM),
            pl.BlockSpec((BLK, F3), lambda i: (i, 0)),
            pl.BlockSpec((F3, N), lambda i: (0, 0)),
        ],
        out_specs=pl.BlockSpec((BLK // 8, N // 128, 8, 128),
                               lambda i: (i, 0, 0, 0)),
        out_shape=jax.ShapeDtypeStruct((N // 8, N // 128, 8, 128), _f32),
    )(param.reshape(1), out, outT)


def _patch_body(conn_ref, outT_hbm, src_hbm, dst_hbm, own_hbm, lwp_hbm,
                *refs):
    outT_f = refs[:F3]
    s_v, d_v, key_v, own_v, lw_v, val_v, lwp_sh, sem = refs[F3:F3 + 8]
    cid = lax.axis_index("c")
    sid = lax.axis_index("s")
    wid = sid * NC + cid
    base = wid * EPW
    seg = E // NS
    pltpu.sync_copy(lwp_hbm.at[pl.ds(sid * seg, seg)],
                    lwp_sh.at[pl.ds(sid * seg, seg)])
    for f in range(F3):
        pltpu.sync_copy(outT_hbm.at[f], outT_f[f])
    pltpu.sync_copy(src_hbm.at[pl.ds(base, EPW)], s_v)
    pltpu.sync_copy(dst_hbm.at[pl.ds(base, EPW)], d_v)
    pltpu.sync_copy(own_hbm.at[wid], own_v)
    for j in range(NCHUNK):
        for c in range(CHUNK // 16):
            sl = pl.ds(j * CHUNK + c * 16, 16)
            s = s_v[sl]
            d = d_v[sl]
            key_v[j, pl.ds(c * 16, 16)] = (
                ((s >> 3) * (N // 128) + (d >> 7)) * 1024
                + (s & 7) * 128 + (d & 127))
    plsc.subcore_barrier()
    cps = [
        pltpu.async_copy(lwp_sh.at[own_v.at[j]],
                         lw_v.at[pl.ds(j * CHUNK, CHUNK)], sem)
        for j in range(NCHUNK)
    ]
    for cp in cps:
        cp.wait()

    def cbody(i, _):
        sl = pl.ds(i * 16, 16)
        s = s_v[sl]
        d = d_v[sl]
        acc = jnp.zeros((16,), _f32)
        for f in range(F3):
            acc = acc + (plsc.load_gather(outT_f[f], [s])
                         * plsc.load_gather(outT_f[f], [d]))
        z = acc + lw_v[sl]
        val_v[sl] = 1.0 / (1.0 + jnp.exp(-z))
        return 0

    lax.fori_loop(0, EPW // 16, cbody, 0)
    cps = [
        pltpu.async_copy(val_v.at[pl.ds(j * CHUNK, CHUNK)],
                         conn_ref.at[key_v.at[j]], sem)
        for j in range(NCHUNK)
    ]
    for cp in cps:
        cp.wait()


@functools.cache
def _sc():
    mesh = plsc.VectorSubcoreMesh(core_axis_name="c", subcore_axis_name="s",
                                  num_cores=NC, num_subcores=NS)
    cp = pltpu.CompilerParams(needs_layout_passes=False)
    own = pl.kernel(
        _own_body,
        out_type=jax.ShapeDtypeStruct((N * N,), _i32),
        mesh=mesh,
        compiler_params=cp,
        scratch_types=[
            pltpu.VMEM((EPW,), _i32), pltpu.VMEM((EPW,), _i32),
            pltpu.VMEM((NCHUNK, CHUNK), _i32), pltpu.VMEM((EPW,), _i32),
            pltpu.SemaphoreType.DMA,
        ],
    )
    conv1 = pl.kernel(
        _conv1_body,
        out_type=(jax.ShapeDtypeStruct((NW, N), _f32),
                  jax.ShapeDtypeStruct((NW, N), _f32)),
        mesh=mesh,
        compiler_params=cp,
        scratch_types=[
            pltpu.VMEM((N,), _f32), pltpu.VMEM((N,), _f32),
            pltpu.VMEM((N,), _f32),
            pltpu.VMEM((EPW,), _i32), pltpu.VMEM((EPW,), _i32),
            pltpu.VMEM((EPW,), _f32),
        ],
    )
    conv2 = pl.kernel(
        _conv2_body,
        out_type=jax.ShapeDtypeStruct((NW, F2, N), _f32),
        mesh=mesh,
        compiler_params=cp,
        scratch_types=[pltpu.VMEM((N,), _f32)] * (2 * F2) + [
            pltpu.VMEM((EPW,), _i32), pltpu.VMEM((EPW,), _i32),
            pltpu.VMEM((EPW,), _f32),
        ],
    )
    acc = pl.kernel(
        _acc_body,
        out_type=(jax.ShapeDtypeStruct((NC, E), _f32),
                  jax.ShapeDtypeStruct((NW, NCHUNK, CHUNK), _i32)),
        mesh=mesh,
        compiler_params=cp,
        scratch_types=[
            pltpu.VMEM_SHARED((E,), _f32),
            pltpu.VMEM((E // NS,), _f32),
            pltpu.VMEM((EPW,), _i32), pltpu.VMEM((EPW,), _i32),
            pltpu.VMEM((EPW,), _f32),
            pltpu.VMEM((NCHUNK, CHUNK), _i32),
            pltpu.VMEM((NCHUNK, CHUNK), _i32),
            pltpu.SemaphoreType.DMA,
        ],
    )
    patch = pl.kernel(
        _patch_body,
        out_type=(),
        mesh=mesh,
        compiler_params=cp,
        scratch_types=[pltpu.VMEM((N,), _f32)] * F3 + [
            pltpu.VMEM((EPW,), _i32), pltpu.VMEM((EPW,), _i32),
            pltpu.VMEM((NCHUNK, CHUNK), _i32),
            pltpu.VMEM((NCHUNK, CHUNK), _i32),
            pltpu.VMEM((EPW,), _f32), pltpu.VMEM((EPW,), _f32),
            pltpu.VMEM_SHARED((E,), _f32),
            pltpu.SemaphoreType.DMA,
        ],
    )
    return own, conv1, conv2, acc, patch


def kernel(x, edge_index, edge_attr, fc11_w, fc11_b, fc21_w, fc21_b,
           c21_rel_w, c21_rel_b, c21_root_w, c22_rel_w, c22_rel_b, c22_root_w):
    own, conv1, conv2, acc, patch = _sc()
    src = edge_index[0]
    dst = edge_index[1]
    param, h1row = _pre(x, fc11_w, fc11_b, fc21_w, fc21_b)
    h1 = h1row.reshape(N)
    tmp = own(src, dst)
    agg1p, degp = conv1(h1, src, dst, edge_attr)
    h2T, invdeg = _mid(agg1p, degp, h1row, c21_rel_w, c21_rel_b, c21_root_w)
    agg2p = conv2(h2T, src, dst, edge_attr)
    accp, ownpe = acc(tmp, src, dst, edge_attr)
    out, outT, lwp = _outk(agg2p, invdeg, h2T, c22_rel_w, c22_rel_b,
                           c22_root_w, accp, param)
    dense = _dense(param, out, outT)
    flat_ref = jax.new_ref(dense.reshape(N * N))
    patch(flat_ref, outT, src, dst, ownpe, lwp)
    conn = (flat_ref[...]
            .reshape(N // 8, N // 128, 8, 128)
            .transpose(0, 2, 1, 3)
            .reshape(1, N, N))
    return conn, param
